# baseline (speedup 1.0000x reference)
import jax
import jax.numpy as jnp
from jax import lax
from jax.experimental import pallas as pl
from jax.experimental.pallas import tpu as pltpu
from jax.experimental.pallas import tpu_sc as plsc

B = 16384
MAXLEN = 200
ATT_LEN = 20
D = 32
N_INT = 1000

NC = 2
NS = 16
L = 16
NW = NC * NS
ROWS_W = B // NW
SUB = 64
NSUB = ROWS_W // SUB
GR = 8
SEQP = SUB + L - GR


def _body(iseq_hbm, mseq_hbm, uid_hbm, tip_hbm, tin_hbm, itp_hbm, itn_hbm,
          user_hbm, item_hbm, from_hbm, to_hbm, tw_hbm,
          out_hbm,
          from_v, to_v, tw_v, iseq_v, mseq_v, tip_v, tin_v,
          uid_v, itp_v, itn_v, idxu_v, idxp_v, idxn_v,
          ustage, pstage, nstage, pooled_v, out_v, sem):
  cid = lax.axis_index("c")
  sid = lax.axis_index("s")
  wid = sid * NC + cid

  pltpu.sync_copy(from_hbm, from_v)
  pltpu.sync_copy(to_hbm, to_v)
  pltpu.sync_copy(tw_hbm, tw_v)

  viota = lax.iota(jnp.int32, L)
  zf = jnp.zeros((L,), jnp.float32)

  def do_sub(h, carry):
    base = wid * ROWS_W + h * SUB

    pltpu.sync_copy(iseq_hbm.at[:, pl.ds(base, SUB)], iseq_v.at[:, 0:SUB])
    pltpu.sync_copy(mseq_hbm.at[:, pl.ds(base, SUB)], mseq_v.at[:, 0:SUB])
    pltpu.sync_copy(tip_hbm.at[pl.ds(base, SUB)], tip_v)
    pltpu.sync_copy(tin_hbm.at[pl.ds(base, SUB)], tin_v)
    pltpu.sync_copy(uid_hbm.at[pl.ds(base, SUB)], uid_v)
    pltpu.sync_copy(itp_hbm.at[pl.ds(base, SUB)], itp_v)
    pltpu.sync_copy(itn_hbm.at[pl.ds(base, SUB)], itn_v)

    for j in range(SUB // L):
      sl = pl.ds(j * L, L)
      idxu_v[sl] = lax.shift_right_logical(uid_v[sl], 2)
      idxp_v[sl] = lax.shift_right_logical(itp_v[sl], 2)
      idxn_v[sl] = lax.shift_right_logical(itn_v[sl], 2)

    cps = [
        pltpu.async_copy(user_hbm.at[idxu_v], ustage, sem),
        pltpu.async_copy(item_hbm.at[idxp_v], pstage, sem),
        pltpu.async_copy(item_hbm.at[idxn_v], nstage, sem),
    ]

    def pool_group(g, pcarry):
      accs = [zf] * (2 * GR)
      for c in range(ATT_LEN):
        iidxv = iseq_v[c, pl.ds(g * GR, L)]
        mvalv = mseq_v[c, pl.ds(g * GR, L)]
        wmv = jnp.where(mvalv != 0, tw_v[c, :], zf)
        for l in range(GR):
          s = iidxv[l]
          f0 = from_v[s, pl.ds(0, L)]
          f1 = from_v[s, pl.ds(L, L)]
          wmb = jnp.broadcast_to(wmv[l], (L,))
          accs[2 * l] = accs[2 * l] + wmb * f0
          accs[2 * l + 1] = accs[2 * l + 1] + wmb * f1
      for l in range(GR):
        r = g * GR + l
        pooled_v[r, pl.ds(0, L)] = accs[2 * l]
        pooled_v[r, pl.ds(L, L)] = accs[2 * l + 1]
      return pcarry

    lax.fori_loop(0, SUB // GR, pool_group, 0)

    for cp in cps:
      cp.wait()

    def dot_group(g, dcarry):
      rows = viota + g * L
      sl = pl.ds(g * L, L)
      tipx = tip_v[sl]
      tinx = tin_v[sl]
      o32u = lax.shift_left(jnp.bitwise_and(uid_v[sl], 3), 5)
      o32p = lax.shift_left(jnp.bitwise_and(itp_v[sl], 3), 5)
      o32n = lax.shift_left(jnp.bitwise_and(itn_v[sl], 3), 5)
      sp = zf
      sn = zf
      lp = zf
      ln = zf
      for k in range(D):
        dvec = jnp.bitwise_and(viota + k, D - 1)
        pv = plsc.load_gather(pooled_v, [rows, dvec])
        sp = sp + pv * plsc.load_gather(to_v, [tipx, dvec])
        sn = sn + pv * plsc.load_gather(to_v, [tinx, dvec])
        u = plsc.load_gather(ustage, [rows, o32u + dvec])
        lp = lp + u * plsc.load_gather(pstage, [rows, o32p + dvec])
        ln = ln + u * plsc.load_gather(nstage, [rows, o32n + dvec])
      o0 = 0.5 * lp + 0.5 * sp
      o1 = 0.5 * ln + 0.5 * sn
      plsc.store_scatter(out_v, [rows, jnp.zeros((L,), jnp.int32)], o0)
      plsc.store_scatter(out_v, [rows, jnp.ones((L,), jnp.int32)], o1)
      return dcarry

    lax.fori_loop(0, SUB // L, dot_group, 0)

    pltpu.sync_copy(out_v, out_hbm.at[pl.ds(base, SUB), :])
    return carry

  lax.fori_loop(0, NSUB, do_sub, 0)


BLKR = 896
BPK = BLKR // 4


def _tr_body_factory(R):
  rtile = (R // 128) * 128
  NB = -(-rtile // BLKR)
  slast = rtile - BLKR
  KMAX = -(-NB // 32)
  if KMAX % 2:
    KMAX += 1
  NPAIR = KMAX // 2

  def body(src_hbm, out_hbm, in_a, in_b, out_a, out_b,
           sem_ia, sem_ib, sem_oa, sem_ob):
    cid = lax.axis_index("c")
    sid = lax.axis_index("s")
    wid = sid * NC + cid
    viota = lax.iota(jnp.int32, L)
    dcols = [jnp.bitwise_and(viota + 16 * v, D - 1) for v in range(8)]
    ocols = [lax.shift_right_logical(viota + 16 * v, 5) for v in range(8)]

    def start_of(j):
      blk = jnp.minimum(wid + 32 * j, NB - 1)
      return pl.multiple_of(jnp.minimum(blk * BLKR, slast), 128)

    def issue_in(j, buf, sem):
      r0 = start_of(j)
      for t in range(D // 8):
        pltpu.async_copy(src_hbm.at[pl.ds(8 * t, 8), pl.ds(r0, BLKR)],
                         buf.at[pl.ds(8 * t, 8), 0:BLKR], sem)

    def wait_in(buf, sem):
      for t in range(D // 8):
        pltpu.make_async_copy(src_hbm.at[pl.ds(0, 8), pl.ds(0, BLKR)],
                              buf.at[pl.ds(8 * t, 8), 0:BLKR], sem).wait()

    def compute(j, buf, obuf):
      @plsc.parallel_loop(0, BPK, unroll=8)
      def pk(p):
        for v in range(8):
          val = plsc.load_gather(buf, [dcols[v], ocols[v] + 4 * p])
          obuf[p, pl.ds(16 * v, L)] = val

    def issue_out(j, obuf, sem):
      p0 = pl.multiple_of(lax.shift_right_logical(start_of(j), 2), 32)
      pltpu.async_copy(obuf, out_hbm.at[pl.ds(p0, BPK), :], sem)

    def wait_out(obuf, sem):
      pltpu.make_async_copy(obuf, out_hbm.at[pl.ds(0, BPK), :], sem).wait()

    issue_in(0, in_a, sem_ia)
    issue_in(1, in_b, sem_ib)

    def pair(k, carry):
      ja = 2 * k
      jb = 2 * k + 1
      wait_in(in_a, sem_ia)

      @pl.when(k > 0)
      def _():
        wait_out(out_a, sem_oa)

      compute(ja, in_a, out_a)
      issue_out(ja, out_a, sem_oa)
      issue_in(jnp.minimum(ja + 2, KMAX - 1), in_a, sem_ia)

      wait_in(in_b, sem_ib)

      @pl.when(k > 0)
      def _():
        wait_out(out_b, sem_ob)

      compute(jb, in_b, out_b)
      issue_out(jb, out_b, sem_ob)
      issue_in(jnp.minimum(jb + 2, KMAX - 1), in_b, sem_ib)
      return carry

    lax.fori_loop(0, NPAIR, pair, 0)
    wait_in(in_a, sem_ia)
    wait_in(in_b, sem_ib)
    wait_out(out_a, sem_oa)
    wait_out(out_b, sem_ob)

  return body


def _transpose_table(srcT, out_packs):
  R = srcT.shape[1]
  mesh = plsc.VectorSubcoreMesh(core_axis_name="c", subcore_axis_name="s")
  f = pl.kernel(
      _tr_body_factory(R),
      out_type=jax.ShapeDtypeStruct((out_packs, 128), jnp.float32),
      mesh=mesh,
      scratch_types=[
          pltpu.VMEM((D, BLKR + 1), jnp.float32),
          pltpu.VMEM((D, BLKR + 1), jnp.float32),
          pltpu.VMEM((BPK, 128), jnp.float32),
          pltpu.VMEM((BPK, 128), jnp.float32),
          pltpu.SemaphoreType.DMA,
          pltpu.SemaphoreType.DMA,
          pltpu.SemaphoreType.DMA,
          pltpu.SemaphoreType.DMA,
      ],
      compiler_params=pltpu.CompilerParams(
          needs_layout_passes=False, use_tc_tiling_on_sc=True),
  )
  return f(srcT)


@jax.jit
def _run(iseq_t, mseq_t, uid, tip, tin, itp, itn,
         user128, item128, from_t, to_t, tw_prep):
  mesh = plsc.VectorSubcoreMesh(core_axis_name="c", subcore_axis_name="s")
  f = pl.kernel(
      _body,
      out_type=jax.ShapeDtypeStruct((B, 2), jnp.float32),
      mesh=mesh,
      scratch_types=[
          pltpu.VMEM((N_INT, D), jnp.float32),
          pltpu.VMEM((N_INT, D), jnp.float32),
          pltpu.VMEM((ATT_LEN, L), jnp.float32),
          pltpu.VMEM((ATT_LEN, SEQP), jnp.int32),
          pltpu.VMEM((ATT_LEN, SEQP), jnp.int32),
          pltpu.VMEM((SUB,), jnp.int32),
          pltpu.VMEM((SUB,), jnp.int32),
          pltpu.VMEM((SUB,), jnp.int32),
          pltpu.VMEM((SUB,), jnp.int32),
          pltpu.VMEM((SUB,), jnp.int32),
          pltpu.VMEM((SUB,), jnp.int32),
          pltpu.VMEM((SUB,), jnp.int32),
          pltpu.VMEM((SUB,), jnp.int32),
          pltpu.VMEM((SUB, 128), jnp.float32),
          pltpu.VMEM((SUB, 128), jnp.float32),
          pltpu.VMEM((SUB, 128), jnp.float32),
          pltpu.VMEM((SUB, D), jnp.float32),
          pltpu.VMEM((SUB, 2), jnp.float32),
          pltpu.SemaphoreType.DMA,
      ],
      compiler_params=pltpu.CompilerParams(
          needs_layout_passes=False, use_tc_tiling_on_sc=False),
  )
  return f(iseq_t, mseq_t, uid, tip, tin, itp, itn,
           user128, item128, from_t, to_t, tw_prep)


def kernel(user_id, item_seq, target_item_pos, target_item_neg,
           intention_seq, target_intention_pos, target_intention_neg,
           user_table, item_table, from_intention_table,
           to_intention_table, timestep_w):
  iseq_t = intention_seq[:, MAXLEN - ATT_LEN:].T
  mseq_t = item_seq[:, MAXLEN - ATT_LEN:].T
  uid = user_id[:, 0]
  tip = target_intention_pos[:, 0]
  tin = target_intention_neg[:, 0]
  itp = target_item_pos[:, 0]
  itn = target_item_neg[:, 0]
  nu = user_table.shape[0]
  ni = item_table.shape[0]
  ut = (nu // 128) * 128
  it = (ni // 128) * 128
  user128 = _transpose_table(user_table.T, nu // 4)
  item128 = _transpose_table(item_table.T, ni // 4)
  user128 = lax.dynamic_update_slice(
      user128, user_table[ut:, :].reshape(-1, 128), (ut // 4, 0))
  item128 = lax.dynamic_update_slice(
      item128, item_table[it:, :].reshape(-1, 128), (it // 4, 0))
  tw_prep = jnp.broadcast_to(timestep_w[::-1][:, None], (ATT_LEN, L))
  return _run(iseq_t, mseq_t, uid, tip, tin, itp, itn,
              user128, item128, from_intention_table,
              to_intention_table, tw_prep)


# speedup vs baseline: 1.9323x; 1.9323x over previous
import jax
import jax.numpy as jnp
from jax import lax
from jax.experimental import pallas as pl
from jax.experimental.pallas import tpu as pltpu
from jax.experimental.pallas import tpu_sc as plsc

B = 16384
MAXLEN = 200
ATT_LEN = 20
D = 32
N_INT = 1000

NC = 2
NS = 16
L = 16
NW = NC * NS
ROWS_W = B // NW
SUB = 64
NSUB = ROWS_W // SUB
GR = 8
SEQP = SUB + L - GR


def _body(iseq_hbm, mseq_hbm, uid_hbm, tip_hbm, tin_hbm, itp_hbm, itn_hbm,
          user_hbm, item_hbm, from_hbm, to_hbm, tw_hbm,
          out_hbm,
          from_v, to_v, tw_v, iseq_v, mseq_v, tip_v, tin_v,
          uid_v, itp_v, itn_v, idxu_v, idxp_v, idxn_v,
          ustage, pstage, nstage, pooled_v, out_v, sem):
  cid = lax.axis_index("c")
  sid = lax.axis_index("s")
  wid = sid * NC + cid

  pltpu.sync_copy(from_hbm, from_v)
  pltpu.sync_copy(to_hbm, to_v)
  pltpu.sync_copy(tw_hbm, tw_v)

  viota = lax.iota(jnp.int32, L)
  zf = jnp.zeros((L,), jnp.float32)

  def do_sub(h, carry):
    base = wid * ROWS_W + h * SUB

    pltpu.sync_copy(iseq_hbm.at[:, pl.ds(base, SUB)], iseq_v.at[:, 0:SUB])
    pltpu.sync_copy(mseq_hbm.at[:, pl.ds(base, SUB)], mseq_v.at[:, 0:SUB])
    pltpu.sync_copy(tip_hbm.at[pl.ds(base, SUB)], tip_v)
    pltpu.sync_copy(tin_hbm.at[pl.ds(base, SUB)], tin_v)
    pltpu.sync_copy(uid_hbm.at[pl.ds(base, SUB)], uid_v)
    pltpu.sync_copy(itp_hbm.at[pl.ds(base, SUB)], itp_v)
    pltpu.sync_copy(itn_hbm.at[pl.ds(base, SUB)], itn_v)

    for j in range(SUB // L):
      sl = pl.ds(j * L, L)
      idxu_v[sl] = lax.shift_right_logical(uid_v[sl], 2)
      idxp_v[sl] = lax.shift_right_logical(itp_v[sl], 2)
      idxn_v[sl] = lax.shift_right_logical(itn_v[sl], 2)

    cps = [
        pltpu.async_copy(user_hbm.at[idxu_v], ustage, sem),
        pltpu.async_copy(item_hbm.at[idxp_v], pstage, sem),
        pltpu.async_copy(item_hbm.at[idxn_v], nstage, sem),
    ]

    def pool_group(g, pcarry):
      accs = [zf] * (2 * GR)
      for c in range(ATT_LEN):
        iidxv = iseq_v[c, pl.ds(g * GR, L)]
        mvalv = mseq_v[c, pl.ds(g * GR, L)]
        wmv = jnp.where(mvalv != 0, tw_v[c, :], zf)
        for l in range(GR):
          s = iidxv[l]
          f0 = from_v[s, pl.ds(0, L)]
          f1 = from_v[s, pl.ds(L, L)]
          wmb = jnp.broadcast_to(wmv[l], (L,))
          accs[2 * l] = accs[2 * l] + wmb * f0
          accs[2 * l + 1] = accs[2 * l + 1] + wmb * f1
      for l in range(GR):
        r = g * GR + l
        pooled_v[r, pl.ds(0, L)] = accs[2 * l]
        pooled_v[r, pl.ds(L, L)] = accs[2 * l + 1]
      return pcarry

    lax.fori_loop(0, SUB // GR, pool_group, 0)

    for cp in cps:
      cp.wait()

    def dot_group(g, dcarry):
      rows = viota + g * L
      sl = pl.ds(g * L, L)
      tipx = tip_v[sl]
      tinx = tin_v[sl]
      o32u = lax.shift_left(jnp.bitwise_and(uid_v[sl], 3), 5)
      o32p = lax.shift_left(jnp.bitwise_and(itp_v[sl], 3), 5)
      o32n = lax.shift_left(jnp.bitwise_and(itn_v[sl], 3), 5)
      sp = zf
      sn = zf
      lp = zf
      ln = zf
      for k in range(D):
        dvec = jnp.bitwise_and(viota + k, D - 1)
        pv = plsc.load_gather(pooled_v, [rows, dvec])
        sp = sp + pv * plsc.load_gather(to_v, [tipx, dvec])
        sn = sn + pv * plsc.load_gather(to_v, [tinx, dvec])
        u = plsc.load_gather(ustage, [rows, o32u + dvec])
        lp = lp + u * plsc.load_gather(pstage, [rows, o32p + dvec])
        ln = ln + u * plsc.load_gather(nstage, [rows, o32n + dvec])
      o0 = 0.5 * lp + 0.5 * sp
      o1 = 0.5 * ln + 0.5 * sn
      plsc.store_scatter(out_v, [rows, jnp.zeros((L,), jnp.int32)], o0)
      plsc.store_scatter(out_v, [rows, jnp.ones((L,), jnp.int32)], o1)
      return dcarry

    lax.fori_loop(0, SUB // L, dot_group, 0)

    pltpu.sync_copy(out_v, out_hbm.at[pl.ds(base, SUB), :])
    return carry

  lax.fori_loop(0, NSUB, do_sub, 0)


BLKR = 896
BPK = BLKR // 4


def _tr_body_factory(R):
  rtile = (R // 128) * 128
  NB = -(-rtile // BLKR)
  slast = rtile - BLKR
  KMAX = -(-NB // 32)
  if KMAX % 2:
    KMAX += 1
  NPAIR = KMAX // 2

  def body(src_hbm, out_hbm, in_a, in_b, out_a, out_b,
           sem_ia, sem_ib, sem_oa, sem_ob):
    cid = lax.axis_index("c")
    sid = lax.axis_index("s")
    wid = sid * NC + cid
    viota = lax.iota(jnp.int32, L)
    dcols = [jnp.bitwise_and(viota + 16 * v, D - 1) for v in range(8)]
    ocols = [lax.shift_right_logical(viota + 16 * v, 5) for v in range(8)]

    def start_of(j):
      blk = jnp.minimum(wid + 32 * j, NB - 1)
      return pl.multiple_of(jnp.minimum(blk * BLKR, slast), 128)

    def issue_in(j, buf, sem):
      r0 = start_of(j)
      for t in range(D // 8):
        pltpu.async_copy(src_hbm.at[pl.ds(8 * t, 8), pl.ds(r0, BLKR)],
                         buf.at[pl.ds(8 * t, 8), 0:BLKR], sem)

    def wait_in(buf, sem):
      for t in range(D // 8):
        pltpu.make_async_copy(src_hbm.at[pl.ds(0, 8), pl.ds(0, BLKR)],
                              buf.at[pl.ds(8 * t, 8), 0:BLKR], sem).wait()

    def compute(j, buf, obuf):
      for half in range(2):
        dvec = viota + 16 * half
        for k in range(L):
          rot = jnp.bitwise_and(viota + k, L - 1)
          prk = lax.shift_right_logical(rot, 2)
          colk = lax.shift_left(jnp.bitwise_and(rot, 3), 5) + dvec

          @plsc.parallel_loop(0, BLKR // L, unroll=4)
          def rq(q):
            ridx = rot + q * L
            pvec = prk + q * 4
            val = plsc.load_gather(buf, [dvec, ridx])
            plsc.store_scatter(obuf, [pvec, colk], val)

    def issue_out(j, obuf, sem):
      p0 = pl.multiple_of(lax.shift_right_logical(start_of(j), 2), 32)
      pltpu.async_copy(obuf, out_hbm.at[pl.ds(p0, BPK), :], sem)

    def wait_out(obuf, sem):
      pltpu.make_async_copy(obuf, out_hbm.at[pl.ds(0, BPK), :], sem).wait()

    issue_in(0, in_a, sem_ia)
    issue_in(1, in_b, sem_ib)

    def pair(k, carry):
      ja = 2 * k
      jb = 2 * k + 1
      wait_in(in_a, sem_ia)

      @pl.when(k > 0)
      def _():
        wait_out(out_a, sem_oa)

      compute(ja, in_a, out_a)
      issue_out(ja, out_a, sem_oa)
      issue_in(jnp.minimum(ja + 2, KMAX - 1), in_a, sem_ia)

      wait_in(in_b, sem_ib)

      @pl.when(k > 0)
      def _():
        wait_out(out_b, sem_ob)

      compute(jb, in_b, out_b)
      issue_out(jb, out_b, sem_ob)
      issue_in(jnp.minimum(jb + 2, KMAX - 1), in_b, sem_ib)
      return carry

    lax.fori_loop(0, NPAIR, pair, 0)
    wait_in(in_a, sem_ia)
    wait_in(in_b, sem_ib)
    wait_out(out_a, sem_oa)
    wait_out(out_b, sem_ob)

  return body


def _transpose_table(srcT, out_packs):
  R = srcT.shape[1]
  mesh = plsc.VectorSubcoreMesh(core_axis_name="c", subcore_axis_name="s")
  f = pl.kernel(
      _tr_body_factory(R),
      out_type=jax.ShapeDtypeStruct((out_packs, 128), jnp.float32),
      mesh=mesh,
      scratch_types=[
          pltpu.VMEM((D, BLKR + 1), jnp.float32),
          pltpu.VMEM((D, BLKR + 1), jnp.float32),
          pltpu.VMEM((BPK, 128), jnp.float32),
          pltpu.VMEM((BPK, 128), jnp.float32),
          pltpu.SemaphoreType.DMA,
          pltpu.SemaphoreType.DMA,
          pltpu.SemaphoreType.DMA,
          pltpu.SemaphoreType.DMA,
      ],
      compiler_params=pltpu.CompilerParams(
          needs_layout_passes=False, use_tc_tiling_on_sc=True),
  )
  return f(srcT)


@jax.jit
def _run(iseq_t, mseq_t, uid, tip, tin, itp, itn,
         user128, item128, from_t, to_t, tw_prep):
  mesh = plsc.VectorSubcoreMesh(core_axis_name="c", subcore_axis_name="s")
  f = pl.kernel(
      _body,
      out_type=jax.ShapeDtypeStruct((B, 2), jnp.float32),
      mesh=mesh,
      scratch_types=[
          pltpu.VMEM((N_INT, D), jnp.float32),
          pltpu.VMEM((N_INT, D), jnp.float32),
          pltpu.VMEM((ATT_LEN, L), jnp.float32),
          pltpu.VMEM((ATT_LEN, SEQP), jnp.int32),
          pltpu.VMEM((ATT_LEN, SEQP), jnp.int32),
          pltpu.VMEM((SUB,), jnp.int32),
          pltpu.VMEM((SUB,), jnp.int32),
          pltpu.VMEM((SUB,), jnp.int32),
          pltpu.VMEM((SUB,), jnp.int32),
          pltpu.VMEM((SUB,), jnp.int32),
          pltpu.VMEM((SUB,), jnp.int32),
          pltpu.VMEM((SUB,), jnp.int32),
          pltpu.VMEM((SUB,), jnp.int32),
          pltpu.VMEM((SUB, 128), jnp.float32),
          pltpu.VMEM((SUB, 128), jnp.float32),
          pltpu.VMEM((SUB, 128), jnp.float32),
          pltpu.VMEM((SUB, D), jnp.float32),
          pltpu.VMEM((SUB, 2), jnp.float32),
          pltpu.SemaphoreType.DMA,
      ],
      compiler_params=pltpu.CompilerParams(
          needs_layout_passes=False, use_tc_tiling_on_sc=False),
  )
  return f(iseq_t, mseq_t, uid, tip, tin, itp, itn,
           user128, item128, from_t, to_t, tw_prep)


def kernel(user_id, item_seq, target_item_pos, target_item_neg,
           intention_seq, target_intention_pos, target_intention_neg,
           user_table, item_table, from_intention_table,
           to_intention_table, timestep_w):
  iseq_t = intention_seq[:, MAXLEN - ATT_LEN:].T
  mseq_t = item_seq[:, MAXLEN - ATT_LEN:].T
  uid = user_id[:, 0]
  tip = target_intention_pos[:, 0]
  tin = target_intention_neg[:, 0]
  itp = target_item_pos[:, 0]
  itn = target_item_neg[:, 0]
  nu = user_table.shape[0]
  ni = item_table.shape[0]
  ut = (nu // 128) * 128
  it = (ni // 128) * 128
  user128 = _transpose_table(user_table.T, nu // 4)
  item128 = _transpose_table(item_table.T, ni // 4)
  user128 = lax.dynamic_update_slice(
      user128, user_table[ut:, :].reshape(-1, 128), (ut // 4, 0))
  item128 = lax.dynamic_update_slice(
      item128, item_table[it:, :].reshape(-1, 128), (it // 4, 0))
  tw_prep = jnp.broadcast_to(timestep_w[::-1][:, None], (ATT_LEN, L))
  return _run(iseq_t, mseq_t, uid, tip, tin, itp, itn,
              user128, item128, from_intention_table,
              to_intention_table, tw_prep)
